# Initial kernel scaffold; baseline (speedup 1.0000x reference)
import jax
import jax.numpy as jnp
from jax.experimental import pallas as pl


def kernel(x, edge_index, num_graphs, W1, b1, Wfc, bfc):
    raise NotImplementedError("write your pallas kernel here")


# speedup vs baseline: 184.1294x; 184.1294x over previous
import functools

import jax
import jax.numpy as jnp
from jax import lax
from jax.experimental import pallas as pl
from jax.experimental.pallas import tpu as pltpu
from jax.experimental.pallas import tpu_sc as plsc

N = 100000
E = 1600000
G_OUT = 2000
F = 50
NP = 100096
ROWS = NP // 128
NC = 2
NS = 16
NW = NC * NS
SLICE = NP // NS
EPW = E // NW
CHUNK = 2000
NITER = EPW // CHUNK


def _zero_fill(vref, nwords):
    def body(i, carry):
        vref[pl.ds(i * 16, 16)] = jnp.zeros((16,), jnp.float32)
        return carry
    lax.fori_loop(0, nwords // 16, body, 0)


def _deg_body(dst_hbm, out_hbm, idx_v, ones_v, zbuf_v, deg_sh):
    c = lax.axis_index("c")
    s = lax.axis_index("s")
    wid = s * NC + c

    def ofill(i, carry):
        ones_v[pl.ds(i * 16, 16)] = jnp.full((16,), 1.0, jnp.float32)
        return carry
    lax.fori_loop(0, CHUNK // 16, ofill, 0)
    _zero_fill(zbuf_v, SLICE)

    sl = pl.ds(s * SLICE, SLICE)
    pltpu.sync_copy(zbuf_v, deg_sh.at[sl])
    plsc.subcore_barrier()

    def body(k, carry):
        base = wid * EPW + k * CHUNK
        pltpu.sync_copy(dst_hbm.at[pl.ds(base, CHUNK)], idx_v)
        pltpu.sync_copy(ones_v, deg_sh.at[idx_v], add=True)
        return carry
    lax.fori_loop(0, NITER, body, 0)

    plsc.subcore_barrier()
    pltpu.sync_copy(deg_sh.at[sl], zbuf_v)
    pltpu.sync_copy(zbuf_v, out_hbm.at[pl.ds(c * NP + s * SLICE, SLICE)])


def _msg_body(src_hbm, dst_hbm, t_hbm, out_hbm, idx_v, val_v, zbuf_v, t_sh, acc_sh):
    c = lax.axis_index("c")
    s = lax.axis_index("s")
    wid = s * NC + c

    sl = pl.ds(s * SLICE, SLICE)
    pltpu.sync_copy(t_hbm.at[sl], zbuf_v)
    pltpu.sync_copy(zbuf_v, t_sh.at[sl])
    _zero_fill(zbuf_v, SLICE)
    pltpu.sync_copy(zbuf_v, acc_sh.at[sl])
    plsc.subcore_barrier()

    def body(k, carry):
        base = wid * EPW + k * CHUNK
        pltpu.sync_copy(src_hbm.at[pl.ds(base, CHUNK)], idx_v)
        pltpu.sync_copy(t_sh.at[idx_v], val_v)
        pltpu.sync_copy(dst_hbm.at[pl.ds(base, CHUNK)], idx_v)
        pltpu.sync_copy(val_v, acc_sh.at[idx_v], add=True)
        return carry
    lax.fori_loop(0, NITER, body, 0)

    plsc.subcore_barrier()
    pltpu.sync_copy(acc_sh.at[sl], zbuf_v)
    pltpu.sync_copy(zbuf_v, out_hbm.at[pl.ds(c * NP + s * SLICE, SLICE)])


def _node_body(p0_ref, p1_ref, x_ref, w1_ref, dinv_ref, t_ref):
    deg = p0_ref[...] + p1_ref[...] + 1.0
    dinv = lax.rsqrt(deg)
    dinv_ref[...] = dinv
    t_ref[...] = x_ref[...] * dinv * w1_ref[0]


def _head_body(a0_ref, a1_ref, t_ref, dinv_ref, wfc_ref, bfc_ref, b1_ref,
               off_ref, out_ref):
    conv = dinv_ref[...] * (a0_ref[...] + a1_ref[...] + t_ref[...])
    xg = conv + b1_ref[0] + off_ref[0]
    out_ref[...] = (jnp.dot(xg, wfc_ref[...], preferred_element_type=jnp.float32)
                    + bfc_ref[...])


@functools.lru_cache(maxsize=1)
def _sc_calls():
    mesh = plsc.VectorSubcoreMesh(core_axis_name="c", subcore_axis_name="s",
                                  num_cores=NC, num_subcores=NS)
    deg_call = pl.kernel(
        _deg_body,
        out_type=jax.ShapeDtypeStruct((2 * NP,), jnp.float32),
        mesh=mesh,
        scratch_types=[
            pltpu.VMEM((CHUNK,), jnp.int32),
            pltpu.VMEM((CHUNK,), jnp.float32),
            pltpu.VMEM((SLICE,), jnp.float32),
            pltpu.VMEM_SHARED((NP,), jnp.float32),
        ],
    )
    msg_call = pl.kernel(
        _msg_body,
        out_type=jax.ShapeDtypeStruct((2 * NP,), jnp.float32),
        mesh=mesh,
        scratch_types=[
            pltpu.VMEM((CHUNK,), jnp.int32),
            pltpu.VMEM((CHUNK,), jnp.float32),
            pltpu.VMEM((SLICE,), jnp.float32),
            pltpu.VMEM_SHARED((NP,), jnp.float32),
            pltpu.VMEM_SHARED((NP,), jnp.float32),
        ],
    )
    return deg_call, msg_call


def kernel(x, edge_index, num_graphs, W1, b1, Wfc, bfc):
    deg_call, msg_call = _sc_calls()
    src = edge_index[0]
    dst = edge_index[1]
    x_pad = jnp.pad(x[:, 0], (0, NP - N)).reshape(ROWS, 128)

    deg_p = deg_call(dst)
    p0 = deg_p[:NP].reshape(ROWS, 128)
    p1 = deg_p[NP:].reshape(ROWS, 128)

    dinv, t = pl.pallas_call(
        _node_body,
        out_shape=[jax.ShapeDtypeStruct((ROWS, 128), jnp.float32)] * 2,
        in_specs=[pl.BlockSpec(memory_space=pltpu.VMEM)] * 3
        + [pl.BlockSpec(memory_space=pltpu.SMEM)],
    )(p0, p1, x_pad, W1.reshape(1))

    acc = msg_call(src, dst, t.reshape(NP))

    a0 = acc[:NP][:N].reshape(G_OUT, F)
    a1 = acc[NP:][:N].reshape(G_OUT, F)
    t50 = t.reshape(NP)[:N].reshape(G_OUT, F)
    dinv50 = dinv.reshape(NP)[:N].reshape(G_OUT, F)
    off = (jnp.asarray(num_graphs, jnp.float32) - G_OUT).reshape(1)

    logits = pl.pallas_call(
        _head_body,
        out_shape=jax.ShapeDtypeStruct((G_OUT, 2), jnp.float32),
        in_specs=[pl.BlockSpec(memory_space=pltpu.VMEM)] * 6
        + [pl.BlockSpec(memory_space=pltpu.SMEM)] * 2,
    )(a0, a1, t50, dinv50, Wfc, bfc.reshape(1, 2), b1.reshape(1), off)

    reg = jnp.zeros((0,), jnp.float32)
    return (logits, reg)


# speedup vs baseline: 212.0435x; 1.1516x over previous
import functools

import jax
import jax.numpy as jnp
from jax import lax
from jax.experimental import pallas as pl
from jax.experimental.pallas import tpu as pltpu
from jax.experimental.pallas import tpu_sc as plsc

N = 100000
E = 1600000
G_OUT = 2000
F = 50
NP = 100096
NC = 2
NS = 16
NW = NC * NS
SLICE = NP // NS
EPW = E // NW
CHUNK = 2000
NITER = EPW // CHUNK
PAIRS = (NITER - 1) // 2


def _zero_fill(vref, nwords):
    def body(i, carry):
        vref[pl.ds(i * 16, 16)] = jnp.zeros((16,), jnp.float32)
        return carry
    lax.fori_loop(0, nwords // 16, body, 0)


MAGIC = 0x5F3759DF


def _rsqrt16(d):
    ii = lax.bitcast_convert_type(d, jnp.int32)
    ii = jnp.full((16,), MAGIC, jnp.int32) - lax.shift_right_logical(
        ii, jnp.full((16,), 1, jnp.int32))
    y = lax.bitcast_convert_type(ii, jnp.float32)
    for _ in range(3):
        y = y * (1.5 - 0.5 * d * y * y)
    return y


def _deg_body(dst_hbm, out_hbm, idx_a, idx_b, ones_v, zbuf_v, sem_a, sem_b,
              deg_sh):
    c = lax.axis_index("c")
    s = lax.axis_index("s")
    wid = s * NC + c
    base_w = wid * EPW

    def ofill(i, carry):
        ones_v[pl.ds(i * 16, 16)] = jnp.full((16,), 1.0, jnp.float32)
        return carry
    lax.fori_loop(0, CHUNK // 16, ofill, 0)
    _zero_fill(zbuf_v, SLICE)

    sl = pl.ds(s * SLICE, SLICE)
    pltpu.sync_copy(zbuf_v, deg_sh.at[sl])
    plsc.subcore_barrier()

    def start(k, buf, sem):
        pltpu.async_copy(dst_hbm.at[pl.ds(base_w + k * CHUNK, CHUNK)], buf, sem)

    def wait(buf, sem):
        pltpu.make_async_copy(dst_hbm.at[pl.ds(base_w, CHUNK)], buf, sem).wait()

    def scat(buf):
        pltpu.sync_copy(ones_v, deg_sh.at[buf], add=True)

    start(0, idx_a, sem_a)

    def body(j, carry):
        start(2 * j + 1, idx_b, sem_b)
        wait(idx_a, sem_a)
        scat(idx_a)
        start(2 * j + 2, idx_a, sem_a)
        wait(idx_b, sem_b)
        scat(idx_b)
        return carry
    lax.fori_loop(0, PAIRS, body, 0)
    wait(idx_a, sem_a)
    scat(idx_a)

    plsc.subcore_barrier()
    pltpu.sync_copy(deg_sh.at[sl], zbuf_v)
    pltpu.sync_copy(zbuf_v, out_hbm.at[pl.ds(c * NP + s * SLICE, SLICE)])


def _msg_body(src_hbm, dst_hbm, x_hbm, degp_hbm, w1_hbm,
              acc_out, t_out, dinv_out,
              src_a, src_b, dst_a, dst_b, val_v, xbuf, p0buf, p1buf,
              tbuf, dinvbuf, zbuf_v, wbuf,
              sem_sa, sem_sb, sem_da, sem_db,
              t_sh, acc_sh):
    c = lax.axis_index("c")
    s = lax.axis_index("s")
    wid = s * NC + c
    base_w = wid * EPW
    sl = pl.ds(s * SLICE, SLICE)

    pltpu.sync_copy(x_hbm.at[sl], xbuf)
    pltpu.sync_copy(degp_hbm.at[pl.ds(s * SLICE, SLICE)], p0buf)
    pltpu.sync_copy(degp_hbm.at[pl.ds(NP + s * SLICE, SLICE)], p1buf)
    pltpu.sync_copy(w1_hbm, wbuf)
    w = wbuf[...]

    def node(i, carry):
        ix = pl.ds(i * 16, 16)
        d = p0buf[ix] + p1buf[ix] + 1.0
        y = _rsqrt16(d)
        dinvbuf[ix] = y
        tbuf[ix] = xbuf[ix] * y * w
        return carry
    lax.fori_loop(0, SLICE // 16, node, 0)

    pltpu.sync_copy(tbuf, t_sh.at[sl])
    _zero_fill(zbuf_v, SLICE)
    pltpu.sync_copy(zbuf_v, acc_sh.at[sl])

    @pl.when(c == 0)
    def _():
        pltpu.sync_copy(tbuf, t_out.at[sl])
        pltpu.sync_copy(dinvbuf, dinv_out.at[sl])

    plsc.subcore_barrier()

    def start(k, sbuf, dbuf, ssem, dsem):
        e = pl.ds(base_w + k * CHUNK, CHUNK)
        pltpu.async_copy(src_hbm.at[e], sbuf, ssem)
        pltpu.async_copy(dst_hbm.at[e], dbuf, dsem)

    def wait(sbuf, dbuf, ssem, dsem):
        e = pl.ds(base_w, CHUNK)
        pltpu.make_async_copy(src_hbm.at[e], sbuf, ssem).wait()
        pltpu.make_async_copy(dst_hbm.at[e], dbuf, dsem).wait()

    def proc(sbuf, dbuf):
        pltpu.sync_copy(t_sh.at[sbuf], val_v)
        pltpu.sync_copy(val_v, acc_sh.at[dbuf], add=True)

    start(0, src_a, dst_a, sem_sa, sem_da)

    def body(j, carry):
        start(2 * j + 1, src_b, dst_b, sem_sb, sem_db)
        wait(src_a, dst_a, sem_sa, sem_da)
        proc(src_a, dst_a)
        start(2 * j + 2, src_a, dst_a, sem_sa, sem_da)
        wait(src_b, dst_b, sem_sb, sem_db)
        proc(src_b, dst_b)
        return carry
    lax.fori_loop(0, PAIRS, body, 0)
    wait(src_a, dst_a, sem_sa, sem_da)
    proc(src_a, dst_a)

    plsc.subcore_barrier()
    pltpu.sync_copy(acc_sh.at[sl], zbuf_v)
    pltpu.sync_copy(zbuf_v, acc_out.at[pl.ds(c * NP + s * SLICE, SLICE)])


def _head_body(a0_ref, a1_ref, t_ref, dinv_ref, wfc_ref, bfc_ref, b1_ref,
               off_ref, out_ref):
    conv = dinv_ref[...] * (a0_ref[...] + a1_ref[...] + t_ref[...])
    xg = conv + b1_ref[0] + off_ref[0]
    out_ref[...] = (jnp.dot(xg, wfc_ref[...], preferred_element_type=jnp.float32)
                    + bfc_ref[...])


@functools.lru_cache(maxsize=1)
def _sc_calls():
    mesh = plsc.VectorSubcoreMesh(core_axis_name="c", subcore_axis_name="s",
                                  num_cores=NC, num_subcores=NS)
    deg_call = pl.kernel(
        _deg_body,
        out_type=jax.ShapeDtypeStruct((2 * NP,), jnp.float32),
        mesh=mesh,
        scratch_types=[
            pltpu.VMEM((CHUNK,), jnp.int32),
            pltpu.VMEM((CHUNK,), jnp.int32),
            pltpu.VMEM((CHUNK,), jnp.float32),
            pltpu.VMEM((SLICE,), jnp.float32),
            pltpu.SemaphoreType.DMA,
            pltpu.SemaphoreType.DMA,
            pltpu.VMEM_SHARED((NP,), jnp.float32),
        ],
    )
    msg_call = pl.kernel(
        _msg_body,
        out_type=[jax.ShapeDtypeStruct((2 * NP,), jnp.float32),
                  jax.ShapeDtypeStruct((NP,), jnp.float32),
                  jax.ShapeDtypeStruct((NP,), jnp.float32)],
        mesh=mesh,
        scratch_types=[
            pltpu.VMEM((CHUNK,), jnp.int32),
            pltpu.VMEM((CHUNK,), jnp.int32),
            pltpu.VMEM((CHUNK,), jnp.int32),
            pltpu.VMEM((CHUNK,), jnp.int32),
            pltpu.VMEM((CHUNK,), jnp.float32),
            pltpu.VMEM((SLICE,), jnp.float32),
            pltpu.VMEM((SLICE,), jnp.float32),
            pltpu.VMEM((SLICE,), jnp.float32),
            pltpu.VMEM((SLICE,), jnp.float32),
            pltpu.VMEM((SLICE,), jnp.float32),
            pltpu.VMEM((SLICE,), jnp.float32),
            pltpu.VMEM((16,), jnp.float32),
            pltpu.SemaphoreType.DMA,
            pltpu.SemaphoreType.DMA,
            pltpu.SemaphoreType.DMA,
            pltpu.SemaphoreType.DMA,
            pltpu.VMEM_SHARED((NP,), jnp.float32),
            pltpu.VMEM_SHARED((NP,), jnp.float32),
        ],
    )
    return deg_call, msg_call


def kernel(x, edge_index, num_graphs, W1, b1, Wfc, bfc):
    deg_call, msg_call = _sc_calls()
    src = edge_index[0]
    dst = edge_index[1]
    x_pad = jnp.pad(x[:, 0], (0, NP - N))
    w1p = jnp.broadcast_to(W1.reshape(1), (16,))

    deg_p = deg_call(dst)
    acc, t, dinv = msg_call(src, dst, x_pad, deg_p, w1p)

    a0 = acc[:NP][:N].reshape(G_OUT, F)
    a1 = acc[NP:][:N].reshape(G_OUT, F)
    t50 = t[:N].reshape(G_OUT, F)
    dinv50 = dinv[:N].reshape(G_OUT, F)
    off = (jnp.asarray(num_graphs, jnp.float32) - G_OUT).reshape(1)

    logits = pl.pallas_call(
        _head_body,
        out_shape=jax.ShapeDtypeStruct((G_OUT, 2), jnp.float32),
        in_specs=[pl.BlockSpec(memory_space=pltpu.VMEM)] * 6
        + [pl.BlockSpec(memory_space=pltpu.SMEM)] * 2,
    )(a0, a1, t50, dinv50, Wfc, bfc.reshape(1, 2), b1.reshape(1), off)

    reg = jnp.zeros((0,), jnp.float32)
    return (logits, reg)


# speedup vs baseline: 270.0237x; 1.2734x over previous
import functools

import jax
import jax.numpy as jnp
from jax import lax
from jax.experimental import pallas as pl
from jax.experimental.pallas import tpu as pltpu
from jax.experimental.pallas import tpu_sc as plsc

N = 100000
E = 1600000
G_OUT = 2000
F = 50
NP = 100096
NC = 2
NS = 16
NW = NC * NS
SLICE = NP // NS
EPW = E // NW
CHUNK = 2000
NITER = EPW // CHUNK
PAIRS = (NITER - 1) // 2


def _zero_fill(vref, nwords):
    def body(i, carry):
        vref[pl.ds(i * 16, 16)] = jnp.zeros((16,), jnp.float32)
        return carry
    lax.fori_loop(0, nwords // 16, body, 0)


MAGIC = 0x5F3759DF


def _rsqrt16(d):
    ii = lax.bitcast_convert_type(d, jnp.int32)
    ii = jnp.full((16,), MAGIC, jnp.int32) - lax.shift_right_logical(
        ii, jnp.full((16,), 1, jnp.int32))
    y = lax.bitcast_convert_type(ii, jnp.float32)
    for _ in range(3):
        y = y * (1.5 - 0.5 * d * y * y)
    return y


def _deg_body(edge_hbm, out_hbm, idx_a, idx_b, ones_v, zbuf_v, sem_a, sem_b,
              deg_sh):
    c = lax.axis_index("c")
    s = lax.axis_index("s")
    wid = s * NC + c
    base_w = wid * EPW

    def ofill(i, carry):
        ones_v[pl.ds(i * 16, 16)] = jnp.full((16,), 1.0, jnp.float32)
        return carry
    lax.fori_loop(0, CHUNK // 16, ofill, 0)
    _zero_fill(zbuf_v, SLICE)

    sl = pl.ds(s * SLICE, SLICE)
    pltpu.sync_copy(zbuf_v, deg_sh.at[sl])
    plsc.subcore_barrier()

    def start(k, buf, sem):
        pltpu.async_copy(edge_hbm.at[pl.ds(E + base_w + k * CHUNK, CHUNK)],
                         buf, sem)

    def wait(buf, sem):
        pltpu.make_async_copy(edge_hbm.at[pl.ds(base_w, CHUNK)], buf,
                              sem).wait()

    def scat(buf):
        pltpu.sync_copy(ones_v, deg_sh.at[buf], add=True)

    start(0, idx_a, sem_a)

    def body(j, carry):
        start(2 * j + 1, idx_b, sem_b)
        wait(idx_a, sem_a)
        scat(idx_a)
        start(2 * j + 2, idx_a, sem_a)
        wait(idx_b, sem_b)
        scat(idx_b)
        return carry
    lax.fori_loop(0, PAIRS, body, 0)
    wait(idx_a, sem_a)
    scat(idx_a)

    plsc.subcore_barrier()
    pltpu.sync_copy(deg_sh.at[sl], zbuf_v)
    pltpu.sync_copy(zbuf_v, out_hbm.at[pl.ds(c * NP + s * SLICE, SLICE)])


LAST = N - (NS - 1) * SLICE


def _msg_body(edge_hbm, x_hbm, degp_hbm, w1_hbm,
              u0_out, u1_out,
              src_a, src_b, dst_a, dst_b, val_v, xbuf, p0buf, p1buf,
              tbuf, dinvbuf, zbuf_v, wbuf,
              sem_sa, sem_sb, sem_da, sem_db,
              t_sh, acc_sh):
    c = lax.axis_index("c")
    s = lax.axis_index("s")
    wid = s * NC + c
    base_w = wid * EPW
    sl = pl.ds(s * SLICE, SLICE)

    pltpu.sync_copy(x_hbm.at[sl], xbuf)
    pltpu.sync_copy(degp_hbm.at[pl.ds(s * SLICE, SLICE)], p0buf)
    pltpu.sync_copy(degp_hbm.at[pl.ds(NP + s * SLICE, SLICE)], p1buf)
    pltpu.sync_copy(w1_hbm, wbuf)
    w = wbuf[...]

    def node(i, carry):
        ix = pl.ds(i * 16, 16)
        d = p0buf[ix] + p1buf[ix] + 1.0
        y = _rsqrt16(d)
        dinvbuf[ix] = y
        tbuf[ix] = xbuf[ix] * y * w
        return carry
    lax.fori_loop(0, SLICE // 16, node, 0)

    pltpu.sync_copy(tbuf, t_sh.at[sl])
    _zero_fill(zbuf_v, SLICE)
    pltpu.sync_copy(zbuf_v, acc_sh.at[sl])
    plsc.subcore_barrier()

    def start(k, sbuf, dbuf, ssem, dsem):
        e = pl.ds(base_w + k * CHUNK, CHUNK)
        pltpu.async_copy(edge_hbm.at[pl.ds(base_w + k * CHUNK, CHUNK)], sbuf, ssem)
        pltpu.async_copy(edge_hbm.at[pl.ds(E + base_w + k * CHUNK, CHUNK)], dbuf, dsem)

    def wait(sbuf, dbuf, ssem, dsem):
        e = pl.ds(base_w, CHUNK)
        pltpu.make_async_copy(edge_hbm.at[e], sbuf, ssem).wait()
        pltpu.make_async_copy(edge_hbm.at[e], dbuf, dsem).wait()

    def proc(sbuf, dbuf):
        pltpu.sync_copy(t_sh.at[sbuf], val_v)
        pltpu.sync_copy(val_v, acc_sh.at[dbuf], add=True)

    start(0, src_a, dst_a, sem_sa, sem_da)

    def body(j, carry):
        start(2 * j + 1, src_b, dst_b, sem_sb, sem_db)
        wait(src_a, dst_a, sem_sa, sem_da)
        proc(src_a, dst_a)
        start(2 * j + 2, src_a, dst_a, sem_sa, sem_da)
        wait(src_b, dst_b, sem_sb, sem_db)
        proc(src_b, dst_b)
        return carry
    lax.fori_loop(0, PAIRS, body, 0)
    wait(src_a, dst_a, sem_sa, sem_da)
    proc(src_a, dst_a)

    plsc.subcore_barrier()
    pltpu.sync_copy(acc_sh.at[sl], p0buf)

    def post(i, carry):
        ix = pl.ds(i * 16, 16)
        p0buf[ix] = dinvbuf[ix] * (p0buf[ix] + 0.5 * tbuf[ix])
        return carry
    lax.fori_loop(0, SLICE // 16, post, 0)

    def wr(out_ref):
        @pl.when(s < NS - 1)
        def _():
            pltpu.sync_copy(p0buf, out_ref.at[sl])

        @pl.when(s == NS - 1)
        def _():
            pltpu.sync_copy(p0buf.at[pl.ds(0, LAST)],
                            out_ref.at[pl.ds((NS - 1) * SLICE, LAST)])

    @pl.when(c == 0)
    def _():
        wr(u0_out)

    @pl.when(c == 1)
    def _():
        wr(u1_out)


def _head_body(u0_ref, u1_ref, wfc_ref, bfc_ref, b1_ref, off_ref, out_ref):
    xg = u0_ref[...] + u1_ref[...] + b1_ref[0] + off_ref[0]
    out_ref[...] = (jnp.dot(xg, wfc_ref[...], preferred_element_type=jnp.float32)
                    + bfc_ref[...])


@functools.lru_cache(maxsize=1)
def _sc_calls():
    mesh = plsc.VectorSubcoreMesh(core_axis_name="c", subcore_axis_name="s",
                                  num_cores=NC, num_subcores=NS)
    deg_call = pl.kernel(
        _deg_body,
        out_type=jax.ShapeDtypeStruct((2 * NP,), jnp.float32),
        mesh=mesh,
        scratch_types=[
            pltpu.VMEM((CHUNK,), jnp.int32),
            pltpu.VMEM((CHUNK,), jnp.int32),
            pltpu.VMEM((CHUNK,), jnp.float32),
            pltpu.VMEM((SLICE,), jnp.float32),
            pltpu.SemaphoreType.DMA,
            pltpu.SemaphoreType.DMA,
            pltpu.VMEM_SHARED((NP,), jnp.float32),
        ],
    )
    msg_call = pl.kernel(
        _msg_body,
        out_type=[jax.ShapeDtypeStruct((N,), jnp.float32),
                  jax.ShapeDtypeStruct((N,), jnp.float32)],
        mesh=mesh,
        scratch_types=[
            pltpu.VMEM((CHUNK,), jnp.int32),
            pltpu.VMEM((CHUNK,), jnp.int32),
            pltpu.VMEM((CHUNK,), jnp.int32),
            pltpu.VMEM((CHUNK,), jnp.int32),
            pltpu.VMEM((CHUNK,), jnp.float32),
            pltpu.VMEM((SLICE,), jnp.float32),
            pltpu.VMEM((SLICE,), jnp.float32),
            pltpu.VMEM((SLICE,), jnp.float32),
            pltpu.VMEM((SLICE,), jnp.float32),
            pltpu.VMEM((SLICE,), jnp.float32),
            pltpu.VMEM((SLICE,), jnp.float32),
            pltpu.VMEM((16,), jnp.float32),
            pltpu.SemaphoreType.DMA,
            pltpu.SemaphoreType.DMA,
            pltpu.SemaphoreType.DMA,
            pltpu.SemaphoreType.DMA,
            pltpu.VMEM_SHARED((NP,), jnp.float32),
            pltpu.VMEM_SHARED((NP,), jnp.float32),
        ],
    )
    return deg_call, msg_call


def kernel(x, edge_index, num_graphs, W1, b1, Wfc, bfc):
    deg_call, msg_call = _sc_calls()
    x_pad = jnp.pad(x.reshape(N), (0, NP - N))
    w1p = jnp.broadcast_to(W1.reshape(1), (16,))

    edge_flat = edge_index.reshape(2 * E)
    deg_p = deg_call(edge_flat)
    u0, u1 = msg_call(edge_flat, x_pad, deg_p, w1p)

    off = (jnp.asarray(num_graphs, jnp.float32) - G_OUT).reshape(1)

    logits = pl.pallas_call(
        _head_body,
        out_shape=jax.ShapeDtypeStruct((G_OUT, 2), jnp.float32),
        in_specs=[pl.BlockSpec(memory_space=pltpu.VMEM)] * 4
        + [pl.BlockSpec(memory_space=pltpu.SMEM)] * 2,
    )(u0.reshape(G_OUT, F), u1.reshape(G_OUT, F), Wfc, bfc.reshape(1, 2),
      b1.reshape(1), off)

    reg = jnp.zeros((0,), jnp.float32)
    return (logits, reg)
